# baseline (speedup 1.0000x reference)
import jax
import jax.numpy as jnp
from jax.experimental import pallas as pl

H = 300
W = 300
TILE = 48
HALO = 5
WIN = TILE + 2 * HALO
NT = -(-H // TILE)
PADH = NT * TILE + 2 * HALO
C = 64
C2 = 2 * C


def _shw(a, off):
    n = a.shape[1]
    zshape = list(a.shape)
    zshape[1] = 1
    z = jnp.zeros(zshape, a.dtype)
    if off == 1:
        return jnp.concatenate([z, jax.lax.slice_in_dim(a, 0, n - 1, axis=1)],
                               axis=1)
    return jnp.concatenate([jax.lax.slice_in_dim(a, 1, n, axis=1), z], axis=1)


def _stencil(z):
    wsum = z + _shw(z, 1) + _shw(z, -1)
    return wsum[1:-1] + z[:-2] + z[2:]


def _body(map_ref, w0_ref, b0_ref, w1_ref, b1_ref, w2_ref, b2_ref,
          w3_ref, b3_ref, psum_ref, pcnt_ref):
    t = pl.program_id(1)
    rows = pl.ds(t * TILE, WIN)
    act_a = (map_ref[0, 0, rows, :] == 1.0).astype(jnp.float32)
    act_b = (map_ref[0, 1, rows, :] == 1.0).astype(jnp.float32)

    def dinv_of(act):
        nbr = (act[:-2] + act[2:] + _shw(act, 1)[1:-1] + _shw(act, -1)[1:-1])
        deg = act[1:-1] * (1.0 + nbr)
        return jnp.where(deg > 0.0, jax.lax.rsqrt(jnp.maximum(deg, 1.0)), 0.0)

    dinv_a = dinv_of(act_a)
    dinv_b = dinv_of(act_b)
    n3 = WIN - 2
    d3 = jnp.concatenate(
        [jnp.broadcast_to(dinv_a[:, :, None], (n3, W, C)),
         jnp.broadcast_to(dinv_b[:, :, None], (n3, W, C))], axis=-1)

    row0 = (t * TILE - HALO + 1).astype(jnp.float32)
    rr3 = row0 + jax.lax.broadcasted_iota(
        jnp.int32, (n3, W, C2), 0).astype(jnp.float32)
    cc3 = jax.lax.broadcasted_iota(
        jnp.int32, (n3, W, C2), 1).astype(jnp.float32)
    v = d3 * (rr3 * w0_ref[0, :][None, None, :] +
              cc3 * w0_ref[1, :][None, None, :])
    h = d3[1:-1] * _stencil(v) + b0_ref[0, :][None, None, :]

    def gcn(x, w_ref, b_ref, k):
        z = d3[k - 1:k - 1 + x.shape[0]] * x
        s = _stencil(z)
        u = d3[k:k + s.shape[0]] * s
        y = jnp.dot(u.reshape(-1, C2), w_ref[...],
                    preferred_element_type=jnp.float32)
        return y.reshape(s.shape) + b_ref[0, :][None, None, :]

    h = jax.nn.relu(gcn(h, w1_ref, b1_ref, 2))
    h = jax.nn.relu(gcn(h, w2_ref, b2_ref, 3))
    h = gcn(h, w3_ref, b3_ref, 4)

    part = jnp.where(d3[HALO - 1:HALO - 1 + TILE] > 0.0, h, 0.0)
    part = part.sum(axis=0).sum(axis=0)
    psum_ref[...] = part.reshape(1, 1, 1, C2)
    cnt_a = act_a[HALO:HALO + TILE, :].sum()
    cnt_b = act_b[HALO:HALO + TILE, :].sum()
    cnt = jnp.concatenate([jnp.broadcast_to(cnt_a.reshape(1), (C,)),
                           jnp.broadcast_to(cnt_b.reshape(1), (C,))])
    pcnt_ref[...] = cnt.reshape(1, 1, 1, C2)


def _blockdiag(w):
    cin, cout = w.shape
    z = jnp.zeros((cin, cout), w.dtype)
    return jnp.concatenate(
        [jnp.concatenate([w, z], axis=1), jnp.concatenate([z, w], axis=1)],
        axis=0)


def kernel(map, W0, b0, W1, b1, W2, b2, W3, b3):
    b = map.shape[0]
    npair = b // 2
    mp = jnp.pad(map[:, :, :, 0], ((0, 0), (HALO, PADH - H - HALO), (0, 0)))
    mp = mp.reshape(npair, 2, PADH, W)
    tile2 = lambda v: jnp.concatenate([v, v]).reshape(1, -1)
    wspec = lambda shape: pl.BlockSpec(shape, lambda s, t: (0, 0))
    psum, pcnt = pl.pallas_call(
        _body,
        grid=(npair, NT),
        in_specs=[
            pl.BlockSpec((1, 2, PADH, W), lambda s, t: (s, 0, 0, 0)),
            wspec((2, C2)), wspec((1, C2)),
            wspec((C2, C2)), wspec((1, C2)),
            wspec((C2, C2)), wspec((1, C2)),
            wspec((C2, C2)), wspec((1, C2)),
        ],
        out_specs=[
            pl.BlockSpec((1, 1, 1, C2), lambda s, t: (s, t, 0, 0)),
            pl.BlockSpec((1, 1, 1, C2), lambda s, t: (s, t, 0, 0)),
        ],
        out_shape=[
            jax.ShapeDtypeStruct((npair, NT, 1, C2), jnp.float32),
            jax.ShapeDtypeStruct((npair, NT, 1, C2), jnp.float32),
        ],
    )(mp,
      jnp.concatenate([W0, W0], axis=1), tile2(b0),
      _blockdiag(W1), tile2(b1),
      _blockdiag(W2), tile2(b2),
      _blockdiag(jnp.pad(W3, ((0, 0), (0, C - W3.shape[1])))),
      tile2(jnp.pad(b3, (0, C - b3.shape[0]))))
    tot = psum.sum(axis=(1, 2))
    total = jnp.stack([tot[:, :32], tot[:, C:C + 32]], axis=1).reshape(b, 32)
    cnt = pcnt[:, :, 0, :].sum(axis=1)[:, ::C].reshape(b)
    pooled = total / jnp.maximum(cnt, 1.0)[:, None]
    return jax.nn.sigmoid(pooled)


# speedup vs baseline: 1.1462x; 1.1462x over previous
import jax
import jax.numpy as jnp
from jax.experimental import pallas as pl

H = 300
W = 300
TILE = 64
HALO = 5
WIN = TILE + 2 * HALO
NT = -(-H // TILE)
PADH = NT * TILE + 2 * HALO
C = 64
C2 = 2 * C


def _shw(a, off):
    n = a.shape[1]
    zshape = list(a.shape)
    zshape[1] = 1
    z = jnp.zeros(zshape, a.dtype)
    if off == 1:
        return jnp.concatenate([z, jax.lax.slice_in_dim(a, 0, n - 1, axis=1)],
                               axis=1)
    return jnp.concatenate([jax.lax.slice_in_dim(a, 1, n, axis=1), z], axis=1)


def _stencil(z):
    wsum = z + _shw(z, 1) + _shw(z, -1)
    return wsum[1:-1] + z[:-2] + z[2:]


def _body(map_ref, w0_ref, b0_ref, w1_ref, b1_ref, w2_ref, b2_ref,
          psum_ref, pcnt_ref):
    t = pl.program_id(1)
    rows = pl.ds(t * TILE, WIN)
    act_a = (map_ref[0, 0, rows, :] == 1.0).astype(jnp.float32)
    act_b = (map_ref[0, 1, rows, :] == 1.0).astype(jnp.float32)

    def dinv_of(act):
        nbr = (act[:-2] + act[2:] + _shw(act, 1)[1:-1] + _shw(act, -1)[1:-1])
        deg = act[1:-1] * (1.0 + nbr)
        return jnp.where(deg > 0.0, jax.lax.rsqrt(jnp.maximum(deg, 1.0)), 0.0)

    dinv_a = dinv_of(act_a)
    dinv_b = dinv_of(act_b)
    n3 = WIN - 2
    d3 = jnp.concatenate(
        [jnp.broadcast_to(dinv_a[:, :, None], (n3, W, C)),
         jnp.broadcast_to(dinv_b[:, :, None], (n3, W, C))], axis=-1)

    row0 = (t * TILE - HALO + 1).astype(jnp.float32)
    rr3 = row0 + jax.lax.broadcasted_iota(
        jnp.int32, (n3, W, C2), 0).astype(jnp.float32)
    cc3 = jax.lax.broadcasted_iota(
        jnp.int32, (n3, W, C2), 1).astype(jnp.float32)
    v = d3 * (rr3 * w0_ref[0, :][None, None, :] +
              cc3 * w0_ref[1, :][None, None, :])
    h = d3[1:-1] * _stencil(v) + b0_ref[0, :][None, None, :]

    def gcn(x, w_ref, b_ref, k):
        z = d3[k - 1:k - 1 + x.shape[0]] * x
        s = _stencil(z)
        u = d3[k:k + s.shape[0]] * s
        y = jnp.dot(u.reshape(-1, C2), w_ref[...],
                    preferred_element_type=jnp.float32)
        return y.reshape(s.shape) + b_ref[0, :][None, None, :]

    h = jax.nn.relu(gcn(h, w1_ref, b1_ref, 2))
    h = jax.nn.relu(gcn(h, w2_ref, b2_ref, 3))

    z = d3[3:3 + h.shape[0]] * h
    s = _stencil(z)
    u = d3[4:4 + TILE] * s
    part = u.sum(axis=0).sum(axis=0)
    psum_ref[...] = part.reshape(1, 1, 1, C2)
    cnt_a = act_a[HALO:HALO + TILE, :].sum()
    cnt_b = act_b[HALO:HALO + TILE, :].sum()
    cnt = jnp.concatenate([jnp.broadcast_to(cnt_a.reshape(1), (C,)),
                           jnp.broadcast_to(cnt_b.reshape(1), (C,))])
    pcnt_ref[...] = cnt.reshape(1, 1, 1, C2)


def _blockdiag(w):
    cin, cout = w.shape
    z = jnp.zeros((cin, cout), w.dtype)
    return jnp.concatenate(
        [jnp.concatenate([w, z], axis=1), jnp.concatenate([z, w], axis=1)],
        axis=0)


def kernel(map, W0, b0, W1, b1, W2, b2, W3, b3):
    b = map.shape[0]
    npair = b // 2
    mp = jnp.pad(map[:, :, :, 0], ((0, 0), (HALO, PADH - H - HALO), (0, 0)))
    mp = mp.reshape(npair, 2, PADH, W)
    tile2 = lambda v: jnp.concatenate([v, v]).reshape(1, -1)
    wspec = lambda shape: pl.BlockSpec(shape, lambda s, t: (0, 0))
    psum, pcnt = pl.pallas_call(
        _body,
        grid=(npair, NT),
        in_specs=[
            pl.BlockSpec((1, 2, PADH, W), lambda s, t: (s, 0, 0, 0)),
            wspec((2, C2)), wspec((1, C2)),
            wspec((C2, C2)), wspec((1, C2)),
            wspec((C2, C2)), wspec((1, C2)),
        ],
        out_specs=[
            pl.BlockSpec((1, 1, 1, C2), lambda s, t: (s, t, 0, 0)),
            pl.BlockSpec((1, 1, 1, C2), lambda s, t: (s, t, 0, 0)),
        ],
        out_shape=[
            jax.ShapeDtypeStruct((npair, NT, 1, C2), jnp.float32),
            jax.ShapeDtypeStruct((npair, NT, 1, C2), jnp.float32),
        ],
    )(mp,
      jnp.concatenate([W0, W0], axis=1), tile2(b0),
      _blockdiag(W1), tile2(b1),
      _blockdiag(W2), tile2(b2))
    tot = psum.sum(axis=(1, 2))
    usum = jnp.stack([tot[:, :C], tot[:, C:]], axis=1).reshape(b, C)
    cnt = pcnt[:, :, 0, :].sum(axis=1)[:, ::C].reshape(b)
    pooled = (usum @ W3 + cnt[:, None] * b3[None, :])
    pooled = pooled / jnp.maximum(cnt, 1.0)[:, None]
    return jax.nn.sigmoid(pooled)
